# baseline (speedup 1.0000x reference)
import functools

import jax
import jax.numpy as jnp
from jax import lax
from jax.experimental import pallas as pl
from jax.experimental.pallas import tpu as pltpu
from jax.experimental.pallas import tpu_sc as plsc

N = 10000
E = 320000
H = 2
HID = 128
OUT = 64

BN = 1024
NB = 10
NPAD = 10240

NC = 2
NS = 16
CH = 128
CPS = 160
NCHUNKP = NS * CPS
EPAD = NCHUNKP * CH
NBUF = 2
IGRP = 4
IBUF = 4
RPS = NPAD // NS
DSTRIPE = NPAD // NS


def _encode_body(h_ref, w_ref, b_ref, o_ref):
    o_ref[0] = jnp.dot(h_ref[...], w_ref[0],
                       preferred_element_type=jnp.float32) + b_ref[0]


def _encode(h, enc_W, enc_b3):
    return pl.pallas_call(
        _encode_body,
        grid=(H, NB),
        in_specs=[
            pl.BlockSpec((BN, HID), lambda i, j: (j, 0)),
            pl.BlockSpec((1, HID, HID), lambda i, j: (i, 0, 0)),
            pl.BlockSpec((1, 1, HID), lambda i, j: (i, 0, 0)),
        ],
        out_specs=pl.BlockSpec((1, BN, HID), lambda i, j: (i, j, 0)),
        out_shape=jax.ShapeDtypeStruct((H, N, HID), jnp.float32),
    )(h, enc_W, enc_b3)


def _sc_agg_body(compute_deg, idx_hbm, xflat_hbm, agg_out, deg_out,
                 ibuf, rows, ones_v, zdeg, agg_sh, deg_sh, *sems):
    isems = sems[:IBUF]
    gsems = sems[IBUF:IBUF + NBUF]
    ssems = sems[IBUF + NBUF:]
    cid = lax.axis_index("c")
    sid = lax.axis_index("s")
    c0 = sid * CPS

    def start_idx(slot, grp):
        pltpu.async_copy(idx_hbm.at[cid, pl.ds(2 * c0 + 8 * grp, 8)],
                         ibuf.at[slot], isems[slot])

    def wait_idx(slot, grp):
        pltpu.make_async_copy(idx_hbm.at[cid, pl.ds(2 * c0 + 8 * grp, 8)],
                              ibuf.at[slot], isems[slot]).wait()

    def start_gather(b, slot, row):
        pltpu.async_copy(xflat_hbm.at[ibuf.at[slot, 2 * row]], rows.at[b],
                         gsems[b])

    def wait_gather(b, slot, row):
        pltpu.make_async_copy(xflat_hbm.at[ibuf.at[slot, 2 * row]],
                              rows.at[b], gsems[b]).wait()

    def start_scatter(rslot, islot, b):
        pltpu.async_copy(rows.at[rslot], agg_sh.at[ibuf.at[islot, 2 * b + 1]],
                         ssems[rslot], add=True)
        if compute_deg:
            pltpu.async_copy(ones_v, deg_sh.at[ibuf.at[islot, 2 * b + 1]],
                             ssems[rslot], add=True)

    def drain_scatter(rslot, islot, b):
        pltpu.make_async_copy(rows.at[rslot],
                              agg_sh.at[ibuf.at[islot, 2 * b + 1]],
                              ssems[rslot]).wait()
        if compute_deg:
            pltpu.make_async_copy(ones_v,
                                  deg_sh.at[ibuf.at[islot, 2 * b + 1]],
                                  ssems[rslot]).wait()

    start_idx(0, 0)
    start_idx(1, 1)

    zk = jnp.zeros((16,), jnp.float32)

    def zero_row(r, _):
        for c in range(HID // 16):
            rows[0, r, pl.ds(c * 16, 16)] = zk
        return 0

    lax.fori_loop(0, CH, zero_row, 0)

    def fill_vec(vec, val):
        k = jnp.full((16,), val, jnp.float32)

        def body(r, _):
            vec[pl.ds(r * 16, 16)] = k
            return 0

        lax.fori_loop(0, vec.shape[0] // 16, body, 0)

    fill_vec(ones_v, 1.0)
    fill_vec(zdeg, 0.0)

    for k in range(RPS // CH):
        pltpu.sync_copy(rows.at[0], agg_sh.at[pl.ds(sid * RPS + k * CH, CH)])
    if compute_deg:
        pltpu.sync_copy(zdeg, deg_sh.at[pl.ds(sid * DSTRIPE, DSTRIPE)])
    plsc.subcore_barrier()

    wait_idx(0, 0)
    start_gather(0, 0, 0)

    NIG = CPS // IGRP
    NS_GR = NIG // IBUF

    def superstep(s, _):
        for q in range(IBUF):
            for b in range(IGRP):
                rslot = b % NBUF
                wait_gather(rslot, q, b)
                start_scatter(rslot, q, b)
                if b > 0:
                    drain_scatter(1 - rslot, q, b - 1)
                elif q > 0:
                    drain_scatter(1 - rslot, q - 1, IGRP - 1)
                else:
                    @pl.when(s > 0)
                    def _():
                        drain_scatter(1 - rslot, IBUF - 1, IGRP - 1)
                if b < IGRP - 1:
                    start_gather(1 - rslot, q, b + 1)
                elif q < IBUF - 1:
                    wait_idx(q + 1, IBUF * s + q + 1)
                    start_gather(1 - rslot, q + 1, 0)
                else:
                    @pl.when(s < NS_GR - 1)
                    def _():
                        wait_idx(0, IBUF * s + q + 1)
                        start_gather(1 - rslot, 0, 0)
            if q < 2:
                start_idx((q + 2) % IBUF, IBUF * s + q + 2)
            else:
                @pl.when(s < NS_GR - 1)
                def _():
                    start_idx((q + 2) % IBUF, IBUF * s + q + 2)
        return 0

    lax.fori_loop(0, NS_GR, superstep, 0)
    drain_scatter(1, IBUF - 1, IGRP - 1)
    plsc.subcore_barrier()

    pltpu.sync_copy(agg_sh.at[pl.ds(sid * RPS, RPS)],
                    agg_out.at[cid, pl.ds(sid * RPS, RPS)])
    if compute_deg:
        pltpu.sync_copy(deg_sh.at[pl.ds(sid * DSTRIPE, DSTRIPE)],
                        deg_out.at[cid, pl.ds(sid * DSTRIPE, DSTRIPE)])


def _sc_agg(xflat, idx3, compute_deg):
    mesh = plsc.VectorSubcoreMesh(core_axis_name="c", subcore_axis_name="s")
    out_type = [jax.ShapeDtypeStruct((H, NPAD, HID), jnp.float32),
                jax.ShapeDtypeStruct((H, NPAD), jnp.float32)]
    scratch = [
        pltpu.VMEM((IBUF, 2 * IGRP, CH), jnp.int32),
        pltpu.VMEM((NBUF, CH, HID), jnp.float32),
        pltpu.VMEM((CH,), jnp.float32),
        pltpu.VMEM((DSTRIPE,), jnp.float32),
        pltpu.VMEM_SHARED((NPAD, HID), jnp.float32),
        pltpu.VMEM_SHARED((NPAD,), jnp.float32),
    ] + [pltpu.SemaphoreType.DMA] * (IBUF + NBUF + 2)
    fn = pl.kernel(
        functools.partial(_sc_agg_body, compute_deg),
        out_type=out_type,
        mesh=mesh,
        scratch_types=scratch,
    )
    return fn(idx3, xflat)


def _combine_body(agg_ref, deg_ref, x_ref, wl_ref, bl_ref, wr_ref, o_ref):
    d = jnp.maximum(deg_ref[0], 1.0)
    a = agg_ref[0] / d
    o_ref[0] = (jnp.dot(a, wl_ref[0], preferred_element_type=jnp.float32)
                + bl_ref[0]
                + jnp.dot(x_ref[0], wr_ref[0],
                          preferred_element_type=jnp.float32))


def _combine(agg, deg3, x, Wl, bl3, Wr):
    return pl.pallas_call(
        _combine_body,
        grid=(H, NB),
        in_specs=[
            pl.BlockSpec((1, BN, HID), lambda i, j: (i, j, 0)),
            pl.BlockSpec((1, BN, 1), lambda i, j: (i, j, 0)),
            pl.BlockSpec((1, BN, HID), lambda i, j: (i, j, 0)),
            pl.BlockSpec((1, HID, HID), lambda i, j: (i, 0, 0)),
            pl.BlockSpec((1, 1, HID), lambda i, j: (i, 0, 0)),
            pl.BlockSpec((1, HID, HID), lambda i, j: (i, 0, 0)),
        ],
        out_specs=pl.BlockSpec((1, BN, HID), lambda i, j: (i, j, 0)),
        out_shape=jax.ShapeDtypeStruct((H, N, HID), jnp.float32),
    )(agg, deg3, x, Wl, bl3, Wr)


def _final_body(agg_ref, deg_ref, x_ref, wl_ref, bl_ref, wr_ref,
                qw_ref, qb_ref, kw_ref, kb_ref, vw_ref, vb_ref,
                beta_ref, pw_ref, pb_ref, o_ref):
    f32 = jnp.float32

    def sage(i):
        d = jnp.maximum(deg_ref[i], 1.0)
        a = agg_ref[i] / d
        return (jnp.dot(a, wl_ref[i], preferred_element_type=f32)
                + bl_ref[i]
                + jnp.dot(x_ref[i], wr_ref[i], preferred_element_type=f32))

    z0 = sage(0)
    z1 = sage(1)
    q0 = jnp.dot(z0, qw_ref[...], preferred_element_type=f32) + qb_ref[0]
    q1 = jnp.dot(z1, qw_ref[...], preferred_element_type=f32) + qb_ref[0]
    k0 = jnp.dot(z0, kw_ref[...], preferred_element_type=f32) + kb_ref[0]
    k1 = jnp.dot(z1, kw_ref[...], preferred_element_type=f32) + kb_ref[0]
    v0 = jnp.dot(z0, vw_ref[...], preferred_element_type=f32) + vb_ref[0]
    v1 = jnp.dot(z1, vw_ref[...], preferred_element_type=f32) + vb_ref[0]

    def soft2(a, b):
        m = jnp.maximum(a, b)
        ea = jnp.exp(a - m)
        eb = jnp.exp(b - m)
        s = ea + eb
        return ea / s, eb / s

    att00 = jnp.sum(q0 * k0, axis=1, keepdims=True)
    att01 = jnp.sum(q0 * k1, axis=1, keepdims=True)
    att10 = jnp.sum(q1 * k0, axis=1, keepdims=True)
    att11 = jnp.sum(q1 * k1, axis=1, keepdims=True)
    a00, a01 = soft2(att00, att01)
    a10, a11 = soft2(att10, att11)
    b = beta_ref[0, 0]
    r0 = b * (a00 * v0 + a01 * v1) + z1
    r1 = b * (a10 * v0 + a11 * v1) + z1
    o_ref[...] = (jnp.dot(r0, pw_ref[0:HID], preferred_element_type=f32)
                  + jnp.dot(r1, pw_ref[HID:2 * HID],
                            preferred_element_type=f32)
                  + pb_ref[0])


def _final(agg, deg3, x, Wl, bl3, Wr,
           Q_W, Q_b2, K_W, K_b2, V_W, V_b2, beta2, P_W, P_b2):
    full = lambda j: (0, 0)
    full3 = lambda j: (0, 0, 0)
    return pl.pallas_call(
        _final_body,
        grid=(NB,),
        in_specs=[
            pl.BlockSpec((H, BN, HID), lambda j: (0, j, 0)),
            pl.BlockSpec((H, BN, 1), lambda j: (0, j, 0)),
            pl.BlockSpec((H, BN, HID), lambda j: (0, j, 0)),
            pl.BlockSpec((H, HID, HID), full3),
            pl.BlockSpec((H, 1, HID), full3),
            pl.BlockSpec((H, HID, HID), full3),
            pl.BlockSpec((HID, HID), full),
            pl.BlockSpec((1, HID), full),
            pl.BlockSpec((HID, HID), full),
            pl.BlockSpec((1, HID), full),
            pl.BlockSpec((HID, HID), full),
            pl.BlockSpec((1, HID), full),
            pl.BlockSpec((1, 1), full),
            pl.BlockSpec((H * HID, OUT), full),
            pl.BlockSpec((1, OUT), full),
        ],
        out_specs=pl.BlockSpec((BN, OUT), lambda j: (j, 0)),
        out_shape=jax.ShapeDtypeStruct((N, OUT), jnp.float32),
    )(agg, deg3, x, Wl, bl3, Wr,
      Q_W, Q_b2, K_W, K_b2, V_W, V_b2, beta2, P_W, P_b2)


def kernel(adj_list, h, enc_W, enc_b, sage_Wl, sage_bl, sage_Wr,
           Q_W, Q_b, K_W, K_b, V_W, V_b, beta, P_W, P_b):
    offs = (jnp.arange(H, dtype=jnp.int32) * N)[:, None]
    pad_iota = jnp.arange(EPAD - E, dtype=jnp.int32)
    pad_src = jnp.broadcast_to(pad_iota % N, (H, EPAD - E))
    pad_dst = jnp.broadcast_to(N + pad_iota % (NPAD - N), (H, EPAD - E))
    src = jnp.concatenate(
        [adj_list[:, 0] + offs, pad_src], axis=1).reshape(H, NCHUNKP, CH)
    dst = jnp.concatenate(
        [adj_list[:, 1], pad_dst], axis=1).reshape(H, NCHUNKP, CH)
    idx3 = jnp.stack([src, dst], axis=2).reshape(H, 2 * NCHUNKP, CH)

    x = _encode(h, enc_W, enc_b.reshape(H, 1, HID))

    agg0, deg = _sc_agg(x.reshape(H * N, HID), idx3, True)
    deg3 = deg.reshape(H, NPAD, 1)
    x = _combine(agg0, deg3, x,
                 sage_Wl[:, 0], sage_bl[:, 0].reshape(H, 1, HID),
                 sage_Wr[:, 0])

    agg1, _ = _sc_agg(x.reshape(H * N, HID), idx3, False)
    return _final(agg1, deg3, x,
                  sage_Wl[:, 1], sage_bl[:, 1].reshape(H, 1, HID),
                  sage_Wr[:, 1],
                  Q_W, Q_b.reshape(1, HID), K_W, K_b.reshape(1, HID),
                  V_W, V_b.reshape(1, HID), beta.reshape(1, 1),
                  P_W, P_b.reshape(1, OUT))


# speedup vs baseline: 1.1630x; 1.1630x over previous
import functools

import jax
import jax.numpy as jnp
from jax import lax
from jax.experimental import pallas as pl
from jax.experimental.pallas import tpu as pltpu
from jax.experimental.pallas import tpu_sc as plsc

N = 10000
E = 320000
H = 2
HID = 128
OUT = 64

BN = 1024
NB = 10
NPAD = 10240

NC = 2
NS = 16
CH = 128
CPS = 160
NCHUNKP = NS * CPS
EPAD = NCHUNKP * CH
NBUF = 2
IGRP = 4
IBUF = 4
RPS = NPAD // NS
DSTRIPE = NPAD // NS


def _encode_body(h_ref, w_ref, b_ref, o_ref):
    o_ref[0] = jnp.dot(h_ref[...], w_ref[0],
                       preferred_element_type=jnp.float32) + b_ref[0]


def _encode(h, enc_W, enc_b3):
    return pl.pallas_call(
        _encode_body,
        grid=(H, NB),
        in_specs=[
            pl.BlockSpec((BN, HID), lambda i, j: (j, 0)),
            pl.BlockSpec((1, HID, HID), lambda i, j: (i, 0, 0)),
            pl.BlockSpec((1, 1, HID), lambda i, j: (i, 0, 0)),
        ],
        out_specs=pl.BlockSpec((1, BN, HID), lambda i, j: (i, j, 0)),
        out_shape=jax.ShapeDtypeStruct((H, N, HID), jnp.float32),
    )(h, enc_W, enc_b3)


def _sc_agg_body(compute_deg, idx_hbm, xflat_hbm, agg_out, deg_out,
                 ibuf, rows, ones_v, zdeg, agg_sh, deg_sh, *sems):
    isems = sems[:IBUF]
    gsems = sems[IBUF:]
    cid = lax.axis_index("c")
    sid = lax.axis_index("s")
    c0 = sid * CPS

    def start_idx(slot, grp):
        pltpu.async_copy(idx_hbm.at[cid, pl.ds(2 * c0 + 8 * grp, 8)],
                         ibuf.at[slot], isems[slot])

    def wait_idx(slot, grp):
        pltpu.make_async_copy(idx_hbm.at[cid, pl.ds(2 * c0 + 8 * grp, 8)],
                              ibuf.at[slot], isems[slot]).wait()

    def start_gather(b, slot, row):
        pltpu.async_copy(xflat_hbm.at[ibuf.at[slot, 2 * row]], rows.at[b],
                         gsems[b])

    def wait_gather(b, slot, row):
        pltpu.make_async_copy(xflat_hbm.at[ibuf.at[slot, 2 * row]],
                              rows.at[b], gsems[b]).wait()

    start_idx(0, 0)
    start_idx(1, 1)

    zk = jnp.zeros((16,), jnp.float32)

    def zero_row(r, _):
        for c in range(HID // 16):
            rows[0, r, pl.ds(c * 16, 16)] = zk
        return 0

    lax.fori_loop(0, CH, zero_row, 0)

    def fill_vec(vec, val):
        k = jnp.full((16,), val, jnp.float32)

        def body(r, _):
            vec[pl.ds(r * 16, 16)] = k
            return 0

        lax.fori_loop(0, vec.shape[0] // 16, body, 0)

    fill_vec(ones_v, 1.0)
    fill_vec(zdeg, 0.0)

    for k in range(RPS // CH):
        pltpu.sync_copy(rows.at[0], agg_sh.at[pl.ds(sid * RPS + k * CH, CH)])
    if compute_deg:
        pltpu.sync_copy(zdeg, deg_sh.at[pl.ds(sid * DSTRIPE, DSTRIPE)])
    plsc.subcore_barrier()

    wait_idx(0, 0)
    start_gather(0, 0, 0)
    start_gather(1, 0, 1)

    NIG = CPS // IGRP
    NS_GR = NIG // IBUF

    def superstep(s, _):
        for q in range(IBUF):
            for b in range(IGRP):
                rslot = b % NBUF
                wait_gather(rslot, q, b)
                pltpu.sync_copy(rows.at[rslot],
                                agg_sh.at[ibuf.at[q, 2 * b + 1]], add=True)
                if compute_deg:
                    pltpu.sync_copy(ones_v,
                                    deg_sh.at[ibuf.at[q, 2 * b + 1]],
                                    add=True)
                if b < IGRP - 2:
                    start_gather(rslot, q, b + 2)
                elif q < IBUF - 1:
                    if b == IGRP - 2:
                        wait_idx(q + 1, IBUF * s + q + 1)
                    start_gather(rslot, q + 1, b - 2)
                else:
                    @pl.when(s < NS_GR - 1)
                    def _():
                        if b == IGRP - 2:
                            wait_idx(0, IBUF * s + q + 1)
                        start_gather(rslot, 0, b - 2)
            if q < 2:
                start_idx((q + 2) % IBUF, IBUF * s + q + 2)
            else:
                @pl.when(s < NS_GR - 1)
                def _():
                    start_idx((q + 2) % IBUF, IBUF * s + q + 2)
        return 0

    lax.fori_loop(0, NS_GR, superstep, 0)
    plsc.subcore_barrier()

    pltpu.sync_copy(agg_sh.at[pl.ds(sid * RPS, RPS)],
                    agg_out.at[cid, pl.ds(sid * RPS, RPS)])
    if compute_deg:
        pltpu.sync_copy(deg_sh.at[pl.ds(sid * DSTRIPE, DSTRIPE)],
                        deg_out.at[cid, pl.ds(sid * DSTRIPE, DSTRIPE)])


def _sc_agg(xflat, idx3, compute_deg):
    mesh = plsc.VectorSubcoreMesh(core_axis_name="c", subcore_axis_name="s")
    out_type = [jax.ShapeDtypeStruct((H, NPAD, HID), jnp.float32),
                jax.ShapeDtypeStruct((H, NPAD), jnp.float32)]
    scratch = [
        pltpu.VMEM((IBUF, 2 * IGRP, CH), jnp.int32),
        pltpu.VMEM((NBUF, CH, HID), jnp.float32),
        pltpu.VMEM((CH,), jnp.float32),
        pltpu.VMEM((DSTRIPE,), jnp.float32),
        pltpu.VMEM_SHARED((NPAD, HID), jnp.float32),
        pltpu.VMEM_SHARED((NPAD,), jnp.float32),
    ] + [pltpu.SemaphoreType.DMA] * (IBUF + NBUF)
    fn = pl.kernel(
        functools.partial(_sc_agg_body, compute_deg),
        out_type=out_type,
        mesh=mesh,
        scratch_types=scratch,
    )
    return fn(idx3, xflat)


def _combine_body(agg_ref, deg_ref, x_ref, wl_ref, bl_ref, wr_ref, o_ref):
    d = jnp.maximum(deg_ref[0], 1.0)
    a = agg_ref[0] / d
    o_ref[0] = (jnp.dot(a, wl_ref[0], preferred_element_type=jnp.float32)
                + bl_ref[0]
                + jnp.dot(x_ref[0], wr_ref[0],
                          preferred_element_type=jnp.float32))


def _combine(agg, deg3, x, Wl, bl3, Wr):
    return pl.pallas_call(
        _combine_body,
        grid=(H, NB),
        in_specs=[
            pl.BlockSpec((1, BN, HID), lambda i, j: (i, j, 0)),
            pl.BlockSpec((1, BN, 1), lambda i, j: (i, j, 0)),
            pl.BlockSpec((1, BN, HID), lambda i, j: (i, j, 0)),
            pl.BlockSpec((1, HID, HID), lambda i, j: (i, 0, 0)),
            pl.BlockSpec((1, 1, HID), lambda i, j: (i, 0, 0)),
            pl.BlockSpec((1, HID, HID), lambda i, j: (i, 0, 0)),
        ],
        out_specs=pl.BlockSpec((1, BN, HID), lambda i, j: (i, j, 0)),
        out_shape=jax.ShapeDtypeStruct((H, N, HID), jnp.float32),
    )(agg, deg3, x, Wl, bl3, Wr)


def _final_body(agg_ref, deg_ref, x_ref, wl_ref, bl_ref, wr_ref,
                qw_ref, qb_ref, kw_ref, kb_ref, vw_ref, vb_ref,
                beta_ref, pw_ref, pb_ref, o_ref):
    f32 = jnp.float32

    def sage(i):
        d = jnp.maximum(deg_ref[i], 1.0)
        a = agg_ref[i] / d
        return (jnp.dot(a, wl_ref[i], preferred_element_type=f32)
                + bl_ref[i]
                + jnp.dot(x_ref[i], wr_ref[i], preferred_element_type=f32))

    z0 = sage(0)
    z1 = sage(1)
    q0 = jnp.dot(z0, qw_ref[...], preferred_element_type=f32) + qb_ref[0]
    q1 = jnp.dot(z1, qw_ref[...], preferred_element_type=f32) + qb_ref[0]
    k0 = jnp.dot(z0, kw_ref[...], preferred_element_type=f32) + kb_ref[0]
    k1 = jnp.dot(z1, kw_ref[...], preferred_element_type=f32) + kb_ref[0]
    v0 = jnp.dot(z0, vw_ref[...], preferred_element_type=f32) + vb_ref[0]
    v1 = jnp.dot(z1, vw_ref[...], preferred_element_type=f32) + vb_ref[0]

    def soft2(a, b):
        m = jnp.maximum(a, b)
        ea = jnp.exp(a - m)
        eb = jnp.exp(b - m)
        s = ea + eb
        return ea / s, eb / s

    att00 = jnp.sum(q0 * k0, axis=1, keepdims=True)
    att01 = jnp.sum(q0 * k1, axis=1, keepdims=True)
    att10 = jnp.sum(q1 * k0, axis=1, keepdims=True)
    att11 = jnp.sum(q1 * k1, axis=1, keepdims=True)
    a00, a01 = soft2(att00, att01)
    a10, a11 = soft2(att10, att11)
    b = beta_ref[0, 0]
    r0 = b * (a00 * v0 + a01 * v1) + z1
    r1 = b * (a10 * v0 + a11 * v1) + z1
    o_ref[...] = (jnp.dot(r0, pw_ref[0:HID], preferred_element_type=f32)
                  + jnp.dot(r1, pw_ref[HID:2 * HID],
                            preferred_element_type=f32)
                  + pb_ref[0])


def _final(agg, deg3, x, Wl, bl3, Wr,
           Q_W, Q_b2, K_W, K_b2, V_W, V_b2, beta2, P_W, P_b2):
    full = lambda j: (0, 0)
    full3 = lambda j: (0, 0, 0)
    return pl.pallas_call(
        _final_body,
        grid=(NB,),
        in_specs=[
            pl.BlockSpec((H, BN, HID), lambda j: (0, j, 0)),
            pl.BlockSpec((H, BN, 1), lambda j: (0, j, 0)),
            pl.BlockSpec((H, BN, HID), lambda j: (0, j, 0)),
            pl.BlockSpec((H, HID, HID), full3),
            pl.BlockSpec((H, 1, HID), full3),
            pl.BlockSpec((H, HID, HID), full3),
            pl.BlockSpec((HID, HID), full),
            pl.BlockSpec((1, HID), full),
            pl.BlockSpec((HID, HID), full),
            pl.BlockSpec((1, HID), full),
            pl.BlockSpec((HID, HID), full),
            pl.BlockSpec((1, HID), full),
            pl.BlockSpec((1, 1), full),
            pl.BlockSpec((H * HID, OUT), full),
            pl.BlockSpec((1, OUT), full),
        ],
        out_specs=pl.BlockSpec((BN, OUT), lambda j: (j, 0)),
        out_shape=jax.ShapeDtypeStruct((N, OUT), jnp.float32),
    )(agg, deg3, x, Wl, bl3, Wr,
      Q_W, Q_b2, K_W, K_b2, V_W, V_b2, beta2, P_W, P_b2)


def kernel(adj_list, h, enc_W, enc_b, sage_Wl, sage_bl, sage_Wr,
           Q_W, Q_b, K_W, K_b, V_W, V_b, beta, P_W, P_b):
    offs = (jnp.arange(H, dtype=jnp.int32) * N)[:, None]
    pad_iota = jnp.arange(EPAD - E, dtype=jnp.int32)
    pad_src = jnp.broadcast_to(pad_iota % N, (H, EPAD - E))
    pad_dst = jnp.broadcast_to(N + pad_iota % (NPAD - N), (H, EPAD - E))
    src = jnp.concatenate(
        [adj_list[:, 0] + offs, pad_src], axis=1).reshape(H, NCHUNKP, CH)
    dst = jnp.concatenate(
        [adj_list[:, 1], pad_dst], axis=1).reshape(H, NCHUNKP, CH)
    idx3 = jnp.stack([src, dst], axis=2).reshape(H, 2 * NCHUNKP, CH)

    x = _encode(h, enc_W, enc_b.reshape(H, 1, HID))

    agg0, deg = _sc_agg(x.reshape(H * N, HID), idx3, True)
    deg3 = deg.reshape(H, NPAD, 1)
    x = _combine(agg0, deg3, x,
                 sage_Wl[:, 0], sage_bl[:, 0].reshape(H, 1, HID),
                 sage_Wr[:, 0])

    agg1, _ = _sc_agg(x.reshape(H * N, HID), idx3, False)
    return _final(agg1, deg3, x,
                  sage_Wl[:, 1], sage_bl[:, 1].reshape(H, 1, HID),
                  sage_Wr[:, 1],
                  Q_W, Q_b.reshape(1, HID), K_W, K_b.reshape(1, HID),
                  V_W, V_b.reshape(1, HID), beta.reshape(1, 1),
                  P_W, P_b.reshape(1, OUT))


# speedup vs baseline: 1.1909x; 1.0240x over previous
import functools

import jax
import jax.numpy as jnp
from jax import lax
from jax.experimental import pallas as pl
from jax.experimental.pallas import tpu as pltpu
from jax.experimental.pallas import tpu_sc as plsc

N = 10000
E = 320000
H = 2
HID = 128
OUT = 64

BN = 1024
NB = 10
NPAD = 10240

NC = 2
NS = 16
CH = 128
CPS = 160
NCHUNKP = NS * CPS
EPAD = NCHUNKP * CH
NBUF = 2
IGRP = 8
RPS = NPAD // NS
DSTRIPE = NPAD // NS


def _encode_body(h_ref, w_ref, b_ref, o_ref):
    o_ref[0] = jnp.dot(h_ref[...], w_ref[0],
                       preferred_element_type=jnp.float32) + b_ref[0]


def _encode(h, enc_W, enc_b3):
    return pl.pallas_call(
        _encode_body,
        grid=(H, NB),
        in_specs=[
            pl.BlockSpec((BN, HID), lambda i, j: (j, 0)),
            pl.BlockSpec((1, HID, HID), lambda i, j: (i, 0, 0)),
            pl.BlockSpec((1, 1, HID), lambda i, j: (i, 0, 0)),
        ],
        out_specs=pl.BlockSpec((1, BN, HID), lambda i, j: (i, j, 0)),
        out_shape=jax.ShapeDtypeStruct((H, N, HID), jnp.float32),
    )(h, enc_W, enc_b3)


def _sc_agg_body(compute_deg, src_hbm, dst_hbm, xflat_hbm, agg_out, deg_out,
                 sbuf, dbuf, rows, ones_v, zdeg, agg_sh, deg_sh, *sems):
    isems = sems[:2]
    gsems = sems[2:]
    cid = lax.axis_index("c")
    sid = lax.axis_index("s")
    c0 = sid * CPS

    def start_idx(slot, grp):
        pltpu.async_copy(src_hbm.at[cid, pl.ds(c0 + IGRP * grp, IGRP)],
                         sbuf.at[slot], isems[slot])
        pltpu.async_copy(dst_hbm.at[cid, pl.ds(c0 + IGRP * grp, IGRP)],
                         dbuf.at[slot], isems[slot])

    def wait_idx(slot, grp):
        pltpu.make_async_copy(src_hbm.at[cid, pl.ds(c0 + IGRP * grp, IGRP)],
                              sbuf.at[slot], isems[slot]).wait()
        pltpu.make_async_copy(dst_hbm.at[cid, pl.ds(c0 + IGRP * grp, IGRP)],
                              dbuf.at[slot], isems[slot]).wait()

    def start_gather(b, slot, row):
        pltpu.async_copy(xflat_hbm.at[sbuf.at[slot, row]], rows.at[b],
                         gsems[b])

    def wait_gather(b, slot, row):
        pltpu.make_async_copy(xflat_hbm.at[sbuf.at[slot, row]],
                              rows.at[b], gsems[b]).wait()

    start_idx(0, 0)
    start_idx(1, 1)

    zk = jnp.zeros((16,), jnp.float32)

    def zero_row(r, _):
        for c in range(HID // 16):
            rows[0, r, pl.ds(c * 16, 16)] = zk
        return 0

    lax.fori_loop(0, CH, zero_row, 0)

    def fill_vec(vec, val):
        k = jnp.full((16,), val, jnp.float32)

        def body(r, _):
            vec[pl.ds(r * 16, 16)] = k
            return 0

        lax.fori_loop(0, vec.shape[0] // 16, body, 0)

    fill_vec(ones_v, 1.0)
    fill_vec(zdeg, 0.0)

    for k in range(RPS // CH):
        pltpu.sync_copy(rows.at[0], agg_sh.at[pl.ds(sid * RPS + k * CH, CH)])
    if compute_deg:
        pltpu.sync_copy(zdeg, deg_sh.at[pl.ds(sid * DSTRIPE, DSTRIPE)])
    plsc.subcore_barrier()

    wait_idx(0, 0)
    start_gather(0, 0, 0)
    start_gather(1, 0, 1)

    NIG = CPS // IGRP
    NS_GR = NIG // 2

    def superstep(s, _):
        for q in range(2):
            for b in range(IGRP):
                rslot = b % NBUF
                wait_gather(rslot, q, b)
                pltpu.sync_copy(rows.at[rslot],
                                agg_sh.at[dbuf.at[q, b]], add=True)
                if compute_deg:
                    pltpu.sync_copy(ones_v,
                                    deg_sh.at[dbuf.at[q, b]],
                                    add=True)
                if b < IGRP - 2:
                    start_gather(rslot, q, b + 2)
                elif q == 0:
                    if b == IGRP - 2:
                        wait_idx(1, 2 * s + 1)
                    start_gather(rslot, 1, b - (IGRP - 2))
                else:
                    @pl.when(s < NS_GR - 1)
                    def _():
                        if b == IGRP - 2:
                            wait_idx(0, 2 * s + 2)
                        start_gather(rslot, 0, b - (IGRP - 2))
            @pl.when(s < NS_GR - 1)
            def _():
                start_idx(q, 2 * s + q + 2)
        return 0

    lax.fori_loop(0, NS_GR, superstep, 0)
    plsc.subcore_barrier()

    pltpu.sync_copy(agg_sh.at[pl.ds(sid * RPS, RPS)],
                    agg_out.at[cid, pl.ds(sid * RPS, RPS)])
    if compute_deg:
        pltpu.sync_copy(deg_sh.at[pl.ds(sid * DSTRIPE, DSTRIPE)],
                        deg_out.at[cid, pl.ds(sid * DSTRIPE, DSTRIPE)])


def _sc_agg(xflat, src3, dst3, compute_deg):
    mesh = plsc.VectorSubcoreMesh(core_axis_name="c", subcore_axis_name="s")
    out_type = [jax.ShapeDtypeStruct((H, NPAD, HID), jnp.float32),
                jax.ShapeDtypeStruct((H, NPAD), jnp.float32)]
    scratch = [
        pltpu.VMEM((2, IGRP, CH), jnp.int32),
        pltpu.VMEM((2, IGRP, CH), jnp.int32),
        pltpu.VMEM((NBUF, CH, HID), jnp.float32),
        pltpu.VMEM((CH,), jnp.float32),
        pltpu.VMEM((DSTRIPE,), jnp.float32),
        pltpu.VMEM_SHARED((NPAD, HID), jnp.float32),
        pltpu.VMEM_SHARED((NPAD,), jnp.float32),
    ] + [pltpu.SemaphoreType.DMA] * (2 + NBUF)
    fn = pl.kernel(
        functools.partial(_sc_agg_body, compute_deg),
        out_type=out_type,
        mesh=mesh,
        scratch_types=scratch,
    )
    return fn(src3, dst3, xflat)


def _combine_body(agg_ref, deg_ref, x_ref, wl_ref, bl_ref, wr_ref, o_ref):
    d = jnp.maximum(deg_ref[0], 1.0)
    a = agg_ref[0] / d
    o_ref[0] = (jnp.dot(a, wl_ref[0], preferred_element_type=jnp.float32)
                + bl_ref[0]
                + jnp.dot(x_ref[0], wr_ref[0],
                          preferred_element_type=jnp.float32))


def _combine(agg, deg3, x, Wl, bl3, Wr):
    return pl.pallas_call(
        _combine_body,
        grid=(H, NB),
        in_specs=[
            pl.BlockSpec((1, BN, HID), lambda i, j: (i, j, 0)),
            pl.BlockSpec((1, BN, 1), lambda i, j: (i, j, 0)),
            pl.BlockSpec((1, BN, HID), lambda i, j: (i, j, 0)),
            pl.BlockSpec((1, HID, HID), lambda i, j: (i, 0, 0)),
            pl.BlockSpec((1, 1, HID), lambda i, j: (i, 0, 0)),
            pl.BlockSpec((1, HID, HID), lambda i, j: (i, 0, 0)),
        ],
        out_specs=pl.BlockSpec((1, BN, HID), lambda i, j: (i, j, 0)),
        out_shape=jax.ShapeDtypeStruct((H, N, HID), jnp.float32),
    )(agg, deg3, x, Wl, bl3, Wr)


def _final_body(agg_ref, deg_ref, x_ref, wl_ref, bl_ref, wr_ref,
                qw_ref, qb_ref, kw_ref, kb_ref, vw_ref, vb_ref,
                beta_ref, pw_ref, pb_ref, o_ref):
    f32 = jnp.float32

    def sage(i):
        d = jnp.maximum(deg_ref[i], 1.0)
        a = agg_ref[i] / d
        return (jnp.dot(a, wl_ref[i], preferred_element_type=f32)
                + bl_ref[i]
                + jnp.dot(x_ref[i], wr_ref[i], preferred_element_type=f32))

    z0 = sage(0)
    z1 = sage(1)
    q0 = jnp.dot(z0, qw_ref[...], preferred_element_type=f32) + qb_ref[0]
    q1 = jnp.dot(z1, qw_ref[...], preferred_element_type=f32) + qb_ref[0]
    k0 = jnp.dot(z0, kw_ref[...], preferred_element_type=f32) + kb_ref[0]
    k1 = jnp.dot(z1, kw_ref[...], preferred_element_type=f32) + kb_ref[0]
    v0 = jnp.dot(z0, vw_ref[...], preferred_element_type=f32) + vb_ref[0]
    v1 = jnp.dot(z1, vw_ref[...], preferred_element_type=f32) + vb_ref[0]

    def soft2(a, b):
        m = jnp.maximum(a, b)
        ea = jnp.exp(a - m)
        eb = jnp.exp(b - m)
        s = ea + eb
        return ea / s, eb / s

    att00 = jnp.sum(q0 * k0, axis=1, keepdims=True)
    att01 = jnp.sum(q0 * k1, axis=1, keepdims=True)
    att10 = jnp.sum(q1 * k0, axis=1, keepdims=True)
    att11 = jnp.sum(q1 * k1, axis=1, keepdims=True)
    a00, a01 = soft2(att00, att01)
    a10, a11 = soft2(att10, att11)
    b = beta_ref[0, 0]
    r0 = b * (a00 * v0 + a01 * v1) + z1
    r1 = b * (a10 * v0 + a11 * v1) + z1
    o_ref[...] = (jnp.dot(r0, pw_ref[0:HID], preferred_element_type=f32)
                  + jnp.dot(r1, pw_ref[HID:2 * HID],
                            preferred_element_type=f32)
                  + pb_ref[0])


def _final(agg, deg3, x, Wl, bl3, Wr,
           Q_W, Q_b2, K_W, K_b2, V_W, V_b2, beta2, P_W, P_b2):
    full = lambda j: (0, 0)
    full3 = lambda j: (0, 0, 0)
    return pl.pallas_call(
        _final_body,
        grid=(NB,),
        in_specs=[
            pl.BlockSpec((H, BN, HID), lambda j: (0, j, 0)),
            pl.BlockSpec((H, BN, 1), lambda j: (0, j, 0)),
            pl.BlockSpec((H, BN, HID), lambda j: (0, j, 0)),
            pl.BlockSpec((H, HID, HID), full3),
            pl.BlockSpec((H, 1, HID), full3),
            pl.BlockSpec((H, HID, HID), full3),
            pl.BlockSpec((HID, HID), full),
            pl.BlockSpec((1, HID), full),
            pl.BlockSpec((HID, HID), full),
            pl.BlockSpec((1, HID), full),
            pl.BlockSpec((HID, HID), full),
            pl.BlockSpec((1, HID), full),
            pl.BlockSpec((1, 1), full),
            pl.BlockSpec((H * HID, OUT), full),
            pl.BlockSpec((1, OUT), full),
        ],
        out_specs=pl.BlockSpec((BN, OUT), lambda j: (j, 0)),
        out_shape=jax.ShapeDtypeStruct((N, OUT), jnp.float32),
    )(agg, deg3, x, Wl, bl3, Wr,
      Q_W, Q_b2, K_W, K_b2, V_W, V_b2, beta2, P_W, P_b2)


def kernel(adj_list, h, enc_W, enc_b, sage_Wl, sage_bl, sage_Wr,
           Q_W, Q_b, K_W, K_b, V_W, V_b, beta, P_W, P_b):
    offs = (jnp.arange(H, dtype=jnp.int32) * N)[:, None]
    pad_iota = jnp.arange(EPAD - E, dtype=jnp.int32)
    pad_src = jnp.broadcast_to(pad_iota % N, (H, EPAD - E))
    pad_dst = jnp.broadcast_to(N + pad_iota % (NPAD - N), (H, EPAD - E))
    src = jnp.concatenate(
        [adj_list[:, 0] + offs, pad_src], axis=1).reshape(H, NCHUNKP, CH)
    dst = jnp.concatenate(
        [adj_list[:, 1], pad_dst], axis=1).reshape(H, NCHUNKP, CH)


    x = _encode(h, enc_W, enc_b.reshape(H, 1, HID))

    agg0, deg = _sc_agg(x.reshape(H * N, HID), src, dst, True)
    deg3 = deg.reshape(H, NPAD, 1)
    x = _combine(agg0, deg3, x,
                 sage_Wl[:, 0], sage_bl[:, 0].reshape(H, 1, HID),
                 sage_Wr[:, 0])

    agg1, _ = _sc_agg(x.reshape(H * N, HID), src, dst, False)
    return _final(agg1, deg3, x,
                  sage_Wl[:, 1], sage_bl[:, 1].reshape(H, 1, HID),
                  sage_Wr[:, 1],
                  Q_W, Q_b.reshape(1, HID), K_W, K_b.reshape(1, HID),
                  V_W, V_b.reshape(1, HID), beta.reshape(1, 1),
                  P_W, P_b.reshape(1, OUT))


# speedup vs baseline: 1.1926x; 1.0014x over previous
import functools

import jax
import jax.numpy as jnp
from jax import lax
from jax.experimental import pallas as pl
from jax.experimental.pallas import tpu as pltpu
from jax.experimental.pallas import tpu_sc as plsc

N = 10000
E = 320000
H = 2
HID = 128
OUT = 64

BN = 1024
NB = 10
NPAD = 10240

NC = 2
NS = 16
CH = 128
CPS = 160
NCHUNKP = NS * CPS
EPAD = NCHUNKP * CH
NBUF = 2
IGRP = 8
RPS = NPAD // NS
DSTRIPE = NPAD // NS


def _encode_body(h_ref, w_ref, b_ref, o_ref):
    o_ref[0] = jnp.dot(h_ref[...], w_ref[0],
                       preferred_element_type=jnp.float32) + b_ref[0]


def _encode(h, enc_W, enc_b3):
    return pl.pallas_call(
        _encode_body,
        grid=(H, NB),
        in_specs=[
            pl.BlockSpec((BN, HID), lambda i, j: (j, 0)),
            pl.BlockSpec((1, HID, HID), lambda i, j: (i, 0, 0)),
            pl.BlockSpec((1, 1, HID), lambda i, j: (i, 0, 0)),
        ],
        out_specs=pl.BlockSpec((1, BN, HID), lambda i, j: (i, j, 0)),
        out_shape=jax.ShapeDtypeStruct((H, N, HID), jnp.float32),
    )(h, enc_W, enc_b3)


def _sc_agg_body(compute_deg, src_hbm, dst_hbm, xflat_hbm, agg_out, deg_out,
                 sbuf, dbuf, rows, ones_v, zdeg, agg_sh, deg_sh, *sems):
    isems = sems[:2]
    gsems = sems[2:]
    cid = lax.axis_index("c")
    sid = lax.axis_index("s")
    c0 = sid * CPS

    def start_idx(slot, grp):
        pltpu.async_copy(src_hbm.at[cid, pl.ds(c0 + IGRP * grp, IGRP)],
                         sbuf.at[slot], isems[slot])
        pltpu.async_copy(dst_hbm.at[cid, pl.ds(c0 + IGRP * grp, IGRP)],
                         dbuf.at[slot], isems[slot])

    def wait_idx(slot, grp):
        pltpu.make_async_copy(src_hbm.at[cid, pl.ds(c0 + IGRP * grp, IGRP)],
                              sbuf.at[slot], isems[slot]).wait()
        pltpu.make_async_copy(dst_hbm.at[cid, pl.ds(c0 + IGRP * grp, IGRP)],
                              dbuf.at[slot], isems[slot]).wait()

    def start_gather(b, slot, row):
        pltpu.async_copy(xflat_hbm.at[sbuf.at[slot, row]], rows.at[b],
                         gsems[b])

    def wait_gather(b, slot, row):
        pltpu.make_async_copy(xflat_hbm.at[sbuf.at[slot, row]],
                              rows.at[b], gsems[b]).wait()

    start_idx(0, 0)
    start_idx(1, 1)

    zk = jnp.zeros((16,), jnp.float32)

    def zero_row(r, _):
        for c in range(HID // 16):
            rows[0, r, pl.ds(c * 16, 16)] = zk
        return 0

    lax.fori_loop(0, CH, zero_row, 0)

    def fill_vec(vec, val):
        k = jnp.full((16,), val, jnp.float32)

        def body(r, _):
            vec[pl.ds(r * 16, 16)] = k
            return 0

        lax.fori_loop(0, vec.shape[0] // 16, body, 0)

    fill_vec(ones_v, 1.0)
    fill_vec(zdeg, 0.0)

    for k in range(RPS // CH):
        pltpu.sync_copy(rows.at[0], agg_sh.at[pl.ds(sid * RPS + k * CH, CH)])
    if compute_deg:
        pltpu.sync_copy(zdeg, deg_sh.at[pl.ds(sid * DSTRIPE, DSTRIPE)])
    plsc.subcore_barrier()

    wait_idx(0, 0)
    start_gather(0, 0, 0)
    start_gather(1, 0, 1)

    NIG = CPS // IGRP
    NS_GR = NIG // 2

    def superstep(s, _):
        for q in range(2):
            for b in range(IGRP):
                rslot = b % NBUF
                wait_gather(rslot, q, b)
                pltpu.sync_copy(rows.at[rslot],
                                agg_sh.at[dbuf.at[q, b]], add=True)
                if compute_deg:
                    pltpu.sync_copy(ones_v,
                                    deg_sh.at[dbuf.at[q, b]],
                                    add=True)
                if b < IGRP - 2:
                    start_gather(rslot, q, b + 2)
                elif q == 0:
                    if b == IGRP - 2:
                        wait_idx(1, 2 * s + 1)
                    start_gather(rslot, 1, b - (IGRP - 2))
                else:
                    @pl.when(s < NS_GR - 1)
                    def _():
                        if b == IGRP - 2:
                            wait_idx(0, 2 * s + 2)
                        start_gather(rslot, 0, b - (IGRP - 2))
            @pl.when(s < NS_GR - 1)
            def _():
                start_idx(q, 2 * s + q + 2)
        return 0

    lax.fori_loop(0, NS_GR, superstep, 0)
    plsc.subcore_barrier()

    pltpu.sync_copy(agg_sh.at[pl.ds(sid * RPS, RPS)],
                    agg_out.at[cid, pl.ds(sid * RPS, RPS)])
    if compute_deg:
        pltpu.sync_copy(deg_sh.at[pl.ds(sid * DSTRIPE, DSTRIPE)],
                        deg_out.at[cid, pl.ds(sid * DSTRIPE, DSTRIPE)])


def _sc_agg(xflat, src3, dst3, compute_deg):
    mesh = plsc.VectorSubcoreMesh(core_axis_name="c", subcore_axis_name="s")
    out_type = [jax.ShapeDtypeStruct((H, NPAD, HID), jnp.float32),
                jax.ShapeDtypeStruct((H, NPAD), jnp.float32)]
    scratch = [
        pltpu.VMEM((2, IGRP, CH), jnp.int32),
        pltpu.VMEM((2, IGRP, CH), jnp.int32),
        pltpu.VMEM((NBUF, CH, HID), jnp.float32),
        pltpu.VMEM((CH,), jnp.float32),
        pltpu.VMEM((DSTRIPE,), jnp.float32),
        pltpu.VMEM_SHARED((NPAD, HID), jnp.float32),
        pltpu.VMEM_SHARED((NPAD,), jnp.float32),
    ] + [pltpu.SemaphoreType.DMA] * (2 + NBUF)
    fn = pl.kernel(
        functools.partial(_sc_agg_body, compute_deg),
        out_type=out_type,
        mesh=mesh,
        scratch_types=scratch,
        compiler_params=pltpu.CompilerParams(use_tc_tiling_on_sc=True),
    )
    return fn(src3, dst3, xflat)


def _combine_body(agg_ref, deg_ref, x_ref, wl_ref, bl_ref, wr_ref, o_ref):
    d = jnp.maximum(deg_ref[0], 1.0)
    a = agg_ref[0] / d
    o_ref[0] = (jnp.dot(a, wl_ref[0], preferred_element_type=jnp.float32)
                + bl_ref[0]
                + jnp.dot(x_ref[0], wr_ref[0],
                          preferred_element_type=jnp.float32))


def _combine(agg, deg3, x, Wl, bl3, Wr):
    return pl.pallas_call(
        _combine_body,
        grid=(H, NB),
        in_specs=[
            pl.BlockSpec((1, BN, HID), lambda i, j: (i, j, 0)),
            pl.BlockSpec((1, BN, 1), lambda i, j: (i, j, 0)),
            pl.BlockSpec((1, BN, HID), lambda i, j: (i, j, 0)),
            pl.BlockSpec((1, HID, HID), lambda i, j: (i, 0, 0)),
            pl.BlockSpec((1, 1, HID), lambda i, j: (i, 0, 0)),
            pl.BlockSpec((1, HID, HID), lambda i, j: (i, 0, 0)),
        ],
        out_specs=pl.BlockSpec((1, BN, HID), lambda i, j: (i, j, 0)),
        out_shape=jax.ShapeDtypeStruct((H, N, HID), jnp.float32),
    )(agg, deg3, x, Wl, bl3, Wr)


def _final_body(agg_ref, deg_ref, x_ref, wl_ref, bl_ref, wr_ref,
                qw_ref, qb_ref, kw_ref, kb_ref, vw_ref, vb_ref,
                beta_ref, pw_ref, pb_ref, o_ref):
    f32 = jnp.float32

    def sage(i):
        d = jnp.maximum(deg_ref[i], 1.0)
        a = agg_ref[i] / d
        return (jnp.dot(a, wl_ref[i], preferred_element_type=f32)
                + bl_ref[i]
                + jnp.dot(x_ref[i], wr_ref[i], preferred_element_type=f32))

    z0 = sage(0)
    z1 = sage(1)
    q0 = jnp.dot(z0, qw_ref[...], preferred_element_type=f32) + qb_ref[0]
    q1 = jnp.dot(z1, qw_ref[...], preferred_element_type=f32) + qb_ref[0]
    k0 = jnp.dot(z0, kw_ref[...], preferred_element_type=f32) + kb_ref[0]
    k1 = jnp.dot(z1, kw_ref[...], preferred_element_type=f32) + kb_ref[0]
    v0 = jnp.dot(z0, vw_ref[...], preferred_element_type=f32) + vb_ref[0]
    v1 = jnp.dot(z1, vw_ref[...], preferred_element_type=f32) + vb_ref[0]

    def soft2(a, b):
        m = jnp.maximum(a, b)
        ea = jnp.exp(a - m)
        eb = jnp.exp(b - m)
        s = ea + eb
        return ea / s, eb / s

    att00 = jnp.sum(q0 * k0, axis=1, keepdims=True)
    att01 = jnp.sum(q0 * k1, axis=1, keepdims=True)
    att10 = jnp.sum(q1 * k0, axis=1, keepdims=True)
    att11 = jnp.sum(q1 * k1, axis=1, keepdims=True)
    a00, a01 = soft2(att00, att01)
    a10, a11 = soft2(att10, att11)
    b = beta_ref[0, 0]
    r0 = b * (a00 * v0 + a01 * v1) + z1
    r1 = b * (a10 * v0 + a11 * v1) + z1
    o_ref[...] = (jnp.dot(r0, pw_ref[0:HID], preferred_element_type=f32)
                  + jnp.dot(r1, pw_ref[HID:2 * HID],
                            preferred_element_type=f32)
                  + pb_ref[0])


def _final(agg, deg3, x, Wl, bl3, Wr,
           Q_W, Q_b2, K_W, K_b2, V_W, V_b2, beta2, P_W, P_b2):
    full = lambda j: (0, 0)
    full3 = lambda j: (0, 0, 0)
    return pl.pallas_call(
        _final_body,
        grid=(NB,),
        in_specs=[
            pl.BlockSpec((H, BN, HID), lambda j: (0, j, 0)),
            pl.BlockSpec((H, BN, 1), lambda j: (0, j, 0)),
            pl.BlockSpec((H, BN, HID), lambda j: (0, j, 0)),
            pl.BlockSpec((H, HID, HID), full3),
            pl.BlockSpec((H, 1, HID), full3),
            pl.BlockSpec((H, HID, HID), full3),
            pl.BlockSpec((HID, HID), full),
            pl.BlockSpec((1, HID), full),
            pl.BlockSpec((HID, HID), full),
            pl.BlockSpec((1, HID), full),
            pl.BlockSpec((HID, HID), full),
            pl.BlockSpec((1, HID), full),
            pl.BlockSpec((1, 1), full),
            pl.BlockSpec((H * HID, OUT), full),
            pl.BlockSpec((1, OUT), full),
        ],
        out_specs=pl.BlockSpec((BN, OUT), lambda j: (j, 0)),
        out_shape=jax.ShapeDtypeStruct((N, OUT), jnp.float32),
    )(agg, deg3, x, Wl, bl3, Wr,
      Q_W, Q_b2, K_W, K_b2, V_W, V_b2, beta2, P_W, P_b2)


def kernel(adj_list, h, enc_W, enc_b, sage_Wl, sage_bl, sage_Wr,
           Q_W, Q_b, K_W, K_b, V_W, V_b, beta, P_W, P_b):
    offs = (jnp.arange(H, dtype=jnp.int32) * N)[:, None]
    pad_iota = jnp.arange(EPAD - E, dtype=jnp.int32)
    pad_src = jnp.broadcast_to(pad_iota % N, (H, EPAD - E))
    pad_dst = jnp.broadcast_to(N + pad_iota % (NPAD - N), (H, EPAD - E))
    src = jnp.concatenate(
        [adj_list[:, 0] + offs, pad_src], axis=1).reshape(H, NCHUNKP, CH)
    dst = jnp.concatenate(
        [adj_list[:, 1], pad_dst], axis=1).reshape(H, NCHUNKP, CH)


    x = _encode(h, enc_W, enc_b.reshape(H, 1, HID))

    agg0, deg = _sc_agg(x.reshape(H * N, HID), src, dst, True)
    deg3 = deg.reshape(H, NPAD, 1)
    x = _combine(agg0, deg3, x,
                 sage_Wl[:, 0], sage_bl[:, 0].reshape(H, 1, HID),
                 sage_Wr[:, 0])

    agg1, _ = _sc_agg(x.reshape(H * N, HID), src, dst, False)
    return _final(agg1, deg3, x,
                  sage_Wl[:, 1], sage_bl[:, 1].reshape(H, 1, HID),
                  sage_Wr[:, 1],
                  Q_W, Q_b.reshape(1, HID), K_W, K_b.reshape(1, HID),
                  V_W, V_b.reshape(1, HID), beta.reshape(1, 1),
                  P_W, P_b.reshape(1, OUT))
